# Initial kernel scaffold; baseline (speedup 1.0000x reference)
import jax
import jax.numpy as jnp
from jax.experimental import pallas as pl


def kernel(input_nodes, input_edges, Wan1, ban1, Wan2, ban2, Wln1, bln1, Wln2, bln2, Wae1, bae1, Wae2, bae2, Wle1, ble1, Wle2, ble2):
    raise NotImplementedError("write your pallas kernel here")


# speedup vs baseline: 98.7600x; 98.7600x over previous
import numpy as np
import jax
import jax.numpy as jnp
from jax.experimental import pallas as pl
from jax.experimental.pallas import tpu as pltpu

_N = 5000
_M = 16
_FN = 16
_FE = 4
_B = 4
_T = 200
_NT = _N // _T
_NE = 79864
_HEAD = _M * _M
_NHEADE = 120


def _head_tables():
    src = np.zeros((_HEAD,), dtype=np.int32)
    msk = np.zeros((_HEAD, 1), dtype=np.float32)
    for i in range(_M):
        k = i * (i - 1) // 2
        for m in range(_M):
            if m >= _M - i:
                src[i * _M + m] = k + m - (_M - i)
                msk[i * _M + m, 0] = 1.0
    out = np.zeros((_NHEADE,), dtype=np.int32)
    e = 0
    for i in range(1, _M):
        for j in range(i):
            out[e] = i * _M + (_M - i) + j
            e += 1
    return src, msk, out


_HEAD_SRC, _HEAD_MSK, _HEAD_OUT = _head_tables()


def _shift_down(x, s):
    return jnp.concatenate([jnp.zeros((s, x.shape[1]), x.dtype), x[:-s, :]], axis=0)


def _graph_body(nA, nB, ed,
                A1, A2, A3, b_an1, Wan2, b_an2,
                E1, E2, b_ae1, Wae2, b_ae2,
                L1a, L1b, b_le1, Wle2, b_le2,
                N1a, N1b, b_ln1, Wln2, b_ln2,
                on_ref, oe_ref):
    f32 = jnp.float32
    T = _T
    i0 = pl.program_id(1) * T

    win = jnp.concatenate([nA[0], nB[0]], axis=0)
    cur = win[_M:_M + T, :]
    ii = i0 + jax.lax.broadcasted_iota(jnp.int32, (T, 1), 0)

    src_all = jnp.concatenate([win[m:m + T, :] for m in range(_M)], axis=0)
    cur_all = jnp.concatenate([cur] * _M, axis=0)
    em_all = ed[0].reshape(_M * T, _FE)

    valid_parts, norme_parts = [], []
    for m in range(_M):
        valid_parts.append((ii + m >= _M).astype(f32))
        cnt = m - jnp.maximum(0, _M - ii)
        norme_parts.append(1.0 / jnp.maximum(cnt, 1).astype(f32))
    valid = jnp.concatenate(valid_parts, axis=0)
    norme = jnp.concatenate(norme_parts, axis=0)

    relu = lambda x: jnp.maximum(x, 0.0)
    dot = lambda a, b: jax.lax.dot_general(
        a, b, (((1,), (0,)), ((), ())), preferred_element_type=f32)

    hn = relu(dot(src_all, A1[...]) + dot(em_all, A2[...])
              + dot(cur_all, A3[...]) + b_an1[...])
    na = relu(dot(hn, Wan2[...]) + b_an2[...]) * valid

    he = relu(dot(src_all, E1[...]) + dot(em_all, E2[...]) + b_ae1[...])
    ea = relu(dot(he, Wae2[...]) + b_ae2[...]) * valid

    pe = _shift_down(ea, T)
    pe = pe + _shift_down(pe, T)
    pe = pe + _shift_down(pe, 2 * T)
    pe = pe + _shift_down(pe, 4 * T)
    pe = pe + _shift_down(pe, 8 * T)
    pe = pe * norme

    g = relu(dot(pe, L1a[...]) + dot(em_all, L1b[...]) + b_le1[...])
    oe_ref[0] = relu(dot(g, Wle2[...]) + b_le2[...]).reshape(_M, T, _FE)

    s = na
    s = s[:8 * T] + s[8 * T:]
    s = s[:4 * T] + s[4 * T:]
    s = s[:2 * T] + s[2 * T:]
    s = s[:T] + s[T:]
    normn = 1.0 / jnp.clip(ii, 1, _M).astype(f32)
    pn = s * normn

    q = relu(dot(pn, N1a[...]) + dot(cur, N1b[...]) + b_ln1[...])
    on_ref[0] = relu(dot(q, Wln2[...]) + b_ln2[...])


def _full_spec(shape):
    return pl.BlockSpec(shape, lambda b, t, nd=len(shape): (0,) * nd)


def kernel(input_nodes, input_edges, Wan1, ban1, Wan2, ban2, Wln1, bln1, Wln2, bln2,
           Wae1, bae1, Wae2, bae2, Wle1, ble1, Wle2, ble2):
    B = input_nodes.shape[0]
    f32 = jnp.float32

    head = input_edges[:, _HEAD_SRC, :] * _HEAD_MSK[None]
    dense = jnp.concatenate([head, input_edges[:, _NHEADE:, :]], axis=1)
    dense = dense.reshape(B, _N, _M, _FE).transpose(0, 2, 1, 3)

    pad_tail = (_NT + 1) * _T - _N - _M
    npad = jnp.concatenate([
        jnp.zeros((B, _M, _FN), f32), input_nodes,
        jnp.zeros((B, pad_tail, _FN), f32)], axis=1)

    weights = (
        Wan1[:_FN], Wan1[_FN:_FN + _FE], Wan1[_FN + _FE:], ban1.reshape(1, -1),
        Wan2, ban2.reshape(1, -1),
        Wae1[:_FN], Wae1[_FN:], bae1.reshape(1, -1),
        Wae2, bae2.reshape(1, -1),
        Wle1[:8], Wle1[8:], ble1.reshape(1, -1),
        Wle2, ble2.reshape(1, -1),
        Wln1[:_FN], Wln1[_FN:], bln1.reshape(1, -1),
        Wln2, bln2.reshape(1, -1),
    )

    in_specs = [
        pl.BlockSpec((1, _T, _FN), lambda b, t: (b, t, 0)),
        pl.BlockSpec((1, _T, _FN), lambda b, t: (b, t + 1, 0)),
        pl.BlockSpec((1, _M, _T, _FE), lambda b, t: (b, 0, t, 0)),
    ] + [_full_spec(w.shape) for w in weights]
    out_specs = [
        pl.BlockSpec((1, _T, _FN), lambda b, t: (b, t, 0)),
        pl.BlockSpec((1, _M, _T, _FE), lambda b, t: (b, 0, t, 0)),
    ]
    out_shape = [
        jax.ShapeDtypeStruct((B, _N, _FN), f32),
        jax.ShapeDtypeStruct((B, _M, _N, _FE), f32),
    ]

    on, oed = pl.pallas_call(
        _graph_body,
        grid=(B, _NT),
        in_specs=in_specs,
        out_specs=out_specs,
        out_shape=out_shape,
        compiler_params=pltpu.CompilerParams(
            dimension_semantics=("parallel", "parallel")),
    )(npad, npad, dense, *weights)

    flat = oed.transpose(0, 2, 1, 3).reshape(B, _N * _M, _FE)
    out_edges = jnp.concatenate([flat[:, _HEAD_OUT, :], flat[:, _HEAD:, :]], axis=1)
    return on, out_edges


# speedup vs baseline: 113.6751x; 1.1510x over previous
import numpy as np
import jax
import jax.numpy as jnp
from jax.experimental import pallas as pl
from jax.experimental.pallas import tpu as pltpu

_N = 5000
_M = 16
_FN = 16
_FE = 4
_B = 4
_T = 200
_NT = _N // _T
_NE = 79864
_HEAD = _M * _M
_NHEADE = 120


def _norm_tables():
    ii = np.arange(_N)[None, :]
    mm = np.arange(_M)[:, None]
    valid = (ii + mm >= _M).astype(np.float32)[:, :, None]
    cnt = mm - np.maximum(0, _M - ii)
    norme = (1.0 / np.maximum(cnt, 1)).astype(np.float32)[:, :, None]
    normn = (1.0 / np.clip(np.arange(_N), 1, _M)).astype(np.float32)[:, None]
    return valid, norme, normn


_VALID, _NORME, _NORMN = _norm_tables()


def _head_tables():
    src = np.zeros((_HEAD,), dtype=np.int32)
    msk = np.zeros((_HEAD, 1), dtype=np.float32)
    for i in range(_M):
        k = i * (i - 1) // 2
        for m in range(_M):
            if m >= _M - i:
                src[i * _M + m] = k + m - (_M - i)
                msk[i * _M + m, 0] = 1.0
    out = np.zeros((_NHEADE,), dtype=np.int32)
    e = 0
    for i in range(1, _M):
        for j in range(i):
            out[e] = i * _M + (_M - i) + j
            e += 1
    return src, msk, out


_HEAD_SRC, _HEAD_MSK, _HEAD_OUT = _head_tables()


def _shift_down(x, s):
    return jnp.concatenate([jnp.zeros((s, x.shape[1]), x.dtype), x[:-s, :]], axis=0)


def _graph_body(nA, nB, ed, vld, nre, nrn,
                W1, b1, W2, b2,
                L1a, L1b, b_le1, Wle2, b_le2,
                N1a, N1b, b_ln1, Wln2, b_ln2,
                on_ref, oe_ref):
    f32 = jnp.float32
    T = _T

    win = jnp.concatenate([nA[0], nB[0]], axis=0)
    cur = win[_M:_M + T, :]

    src_all = jnp.concatenate([win[m:m + T, :] for m in range(_M)], axis=0)
    cur_all = jnp.concatenate([cur] * _M, axis=0)
    em_all = ed[0].reshape(_M * T, _FE)

    valid = vld[...].reshape(_M * T, 1)
    norme = nre[...].reshape(_M * T, 1)
    normn = nrn[...]

    relu = lambda x: jnp.maximum(x, 0.0)
    dot = lambda a, b: jax.lax.dot_general(
        a, b, (((1,), (0,)), ((), ())), preferred_element_type=f32)

    X = jnp.concatenate([src_all, em_all, cur_all], axis=1)
    h = relu(dot(X, W1[...]) + b1[...])
    z = relu(dot(h, W2[...]) + b2[...]) * valid

    pe = _shift_down(z, T)
    pe = pe + _shift_down(pe, T)
    pe = pe + _shift_down(pe, 2 * T)
    pe = pe + _shift_down(pe, 4 * T)
    pe = pe + _shift_down(pe, 8 * T)
    pe = pe * norme

    g = relu(dot(pe, L1a[...]) + dot(em_all, L1b[...]) + b_le1[...])
    oe_ref[0] = relu(dot(g, Wle2[...]) + b_le2[...]).reshape(_M, T, _FE)

    s = z
    s = s[:8 * T] + s[8 * T:]
    s = s[:4 * T] + s[4 * T:]
    s = s[:2 * T] + s[2 * T:]
    s = s[:T] + s[T:]
    pn = s * normn

    q = relu(dot(pn, N1a[...]) + dot(cur, N1b[...]) + b_ln1[...])
    on_ref[0] = relu(dot(q, Wln2[...]) + b_ln2[...])


def _full_spec(shape):
    return pl.BlockSpec(shape, lambda b, t, nd=len(shape): (0,) * nd)


def kernel(input_nodes, input_edges, Wan1, ban1, Wan2, ban2, Wln1, bln1, Wln2, bln2,
           Wae1, bae1, Wae2, bae2, Wle1, ble1, Wle2, ble2):
    B = input_nodes.shape[0]
    f32 = jnp.float32

    head = input_edges[:, _HEAD_SRC, :] * _HEAD_MSK[None]
    dense = jnp.concatenate([head, input_edges[:, _NHEADE:, :]], axis=1)
    dense = dense.reshape(B, _N, _M, _FE).transpose(0, 2, 1, 3)

    pad_tail = (_NT + 1) * _T - _N - _M
    npad = jnp.concatenate([
        jnp.zeros((B, _M, _FN), f32), input_nodes,
        jnp.zeros((B, pad_tail, _FN), f32)], axis=1)

    W1 = jnp.concatenate(
        [Wan1, jnp.concatenate([Wae1, jnp.zeros((_FN, 2 * 8), f32)], axis=0)], axis=1)
    b1 = jnp.concatenate([ban1, bae1]).reshape(1, -1)
    W2 = jnp.concatenate([
        jnp.concatenate([Wan2, jnp.zeros((2 * _FN, 8), f32)], axis=1),
        jnp.concatenate([jnp.zeros((2 * 8, _FN), f32), Wae2], axis=1)], axis=0)
    b2 = jnp.concatenate([ban2, bae2]).reshape(1, -1)
    L1a = jnp.concatenate([jnp.zeros((_FN, 12), f32), Wle1[:8]], axis=0)
    N1a = jnp.concatenate([Wln1[:_FN], jnp.zeros((8, 2 * _FN), f32)], axis=0)

    weights = (
        W1, b1, W2, b2,
        L1a, Wle1[8:], ble1.reshape(1, -1),
        Wle2, ble2.reshape(1, -1),
        N1a, Wln1[_FN:], bln1.reshape(1, -1),
        Wln2, bln2.reshape(1, -1),
    )

    in_specs = [
        pl.BlockSpec((1, _T, _FN), lambda b, t: (b, t, 0)),
        pl.BlockSpec((1, _T, _FN), lambda b, t: (b, t + 1, 0)),
        pl.BlockSpec((1, _M, _T, _FE), lambda b, t: (b, 0, t, 0)),
        pl.BlockSpec((_M, _T, 1), lambda b, t: (0, t, 0)),
        pl.BlockSpec((_M, _T, 1), lambda b, t: (0, t, 0)),
        pl.BlockSpec((_T, 1), lambda b, t: (t, 0)),
    ] + [_full_spec(w.shape) for w in weights]
    out_specs = [
        pl.BlockSpec((1, _T, _FN), lambda b, t: (b, t, 0)),
        pl.BlockSpec((1, _M, _T, _FE), lambda b, t: (b, 0, t, 0)),
    ]
    out_shape = [
        jax.ShapeDtypeStruct((B, _N, _FN), f32),
        jax.ShapeDtypeStruct((B, _M, _N, _FE), f32),
    ]

    on, oed = pl.pallas_call(
        _graph_body,
        grid=(B, _NT),
        in_specs=in_specs,
        out_specs=out_specs,
        out_shape=out_shape,
        compiler_params=pltpu.CompilerParams(
            dimension_semantics=("parallel", "parallel")),
    )(npad, npad, dense, jnp.asarray(_VALID), jnp.asarray(_NORME),
      jnp.asarray(_NORMN), *weights)

    flat = oed.transpose(0, 2, 1, 3).reshape(B, _N * _M, _FE)
    out_edges = jnp.concatenate([flat[:, _HEAD_OUT, :], flat[:, _HEAD:, :]], axis=1)
    return on, out_edges


# speedup vs baseline: 119.2585x; 1.0491x over previous
import numpy as np
import jax
import jax.numpy as jnp
from jax.experimental import pallas as pl
from jax.experimental.pallas import tpu as pltpu

_N = 5000
_M = 16
_FN = 16
_FE = 4
_B = 4
_T = 200
_NT = _N // _T
_NE = 79864
_HEAD = _M * _M
_NHEADE = 120


def _norm_tables():
    ii = np.arange(_N)[None, :]
    mm = np.arange(_M)[:, None]
    valid = (ii + mm >= _M).astype(np.float32)[:, :, None]
    cnt = mm - np.maximum(0, _M - ii)
    norme = (1.0 / np.maximum(cnt, 1)).astype(np.float32)[:, :, None]
    normn = (1.0 / np.clip(np.arange(_N), 1, _M)).astype(np.float32)[:, None]
    return valid, norme, normn


_VALID, _NORME, _NORMN = _norm_tables()


def _head_tables():
    src = np.zeros((_HEAD,), dtype=np.int32)
    msk = np.zeros((_HEAD, 1), dtype=np.float32)
    for i in range(_M):
        k = i * (i - 1) // 2
        for m in range(_M):
            if m >= _M - i:
                src[i * _M + m] = k + m - (_M - i)
                msk[i * _M + m, 0] = 1.0
    out = np.zeros((_NHEADE,), dtype=np.int32)
    e = 0
    for i in range(1, _M):
        for j in range(i):
            out[e] = i * _M + (_M - i) + j
            e += 1
    return src, msk, out


_HEAD_SRC, _HEAD_MSK, _HEAD_OUT = _head_tables()


def _shift_down(x, s):
    return jnp.concatenate([jnp.zeros((s, x.shape[1]), x.dtype), x[:-s, :]], axis=0)


def _graph_body(nA, nB, ed, vld, nre, nrn,
                W1, b1, W2, b2,
                L1a, L1b, b_le1, Wle2, b_le2,
                N1a, N1b, b_ln1, Wln2, b_ln2,
                on_ref, oe_ref):
    f32 = jnp.float32
    T = _T

    win = jnp.concatenate([nA[0], nB[0]], axis=0)
    cur = win[_M:_M + T, :]

    src_all = jnp.concatenate([win[m:m + T, :] for m in range(_M)], axis=0)
    cur_all = jnp.concatenate([cur] * _M, axis=0)
    em_all = ed[0].reshape(_M * T, _FE)

    valid = vld[...].reshape(_M * T, 1)
    norme = nre[...].reshape(_M * T, 1)
    normn = nrn[...]

    relu = lambda x: jnp.maximum(x, 0.0)
    dot = lambda a, b: jax.lax.dot_general(
        a, b, (((1,), (0,)), ((), ())), preferred_element_type=f32)

    X = jnp.concatenate([src_all, em_all, cur_all], axis=1)
    h = relu(dot(X, W1[...]) + b1[...]).astype(jnp.bfloat16)
    z = relu(dot(h, W2[...]) + b2[...]) * valid

    pe = _shift_down(z, T)
    pe = pe + _shift_down(pe, T)
    pe = pe + _shift_down(pe, 2 * T)
    pe = pe + _shift_down(pe, 4 * T)
    pe = pe + _shift_down(pe, 8 * T)
    pe = (pe * norme).astype(jnp.bfloat16)

    g = relu(dot(pe, L1a[...]) + dot(em_all, L1b[...]) + b_le1[...]).astype(jnp.bfloat16)
    oe_ref[0] = relu(dot(g, Wle2[...]) + b_le2[...]).reshape(_M, T, _FE)

    s = z
    s = s[:8 * T] + s[8 * T:]
    s = s[:4 * T] + s[4 * T:]
    s = s[:2 * T] + s[2 * T:]
    s = s[:T] + s[T:]
    pn = (s * normn).astype(jnp.bfloat16)

    q = relu(dot(pn, N1a[...]) + dot(cur, N1b[...]) + b_ln1[...]).astype(jnp.bfloat16)
    on_ref[0] = relu(dot(q, Wln2[...]) + b_ln2[...])


def _full_spec(shape):
    return pl.BlockSpec(shape, lambda b, t, nd=len(shape): (0,) * nd)


def kernel(input_nodes, input_edges, Wan1, ban1, Wan2, ban2, Wln1, bln1, Wln2, bln2,
           Wae1, bae1, Wae2, bae2, Wle1, ble1, Wle2, ble2):
    B = input_nodes.shape[0]
    f32 = jnp.float32

    head = input_edges[:, _HEAD_SRC, :] * _HEAD_MSK[None]
    dense = jnp.concatenate([head, input_edges[:, _NHEADE:, :]], axis=1)
    dense = dense.reshape(B, _N, _M, _FE).transpose(0, 2, 1, 3).astype(jnp.bfloat16)

    pad_tail = (_NT + 1) * _T - _N - _M
    bf16 = jnp.bfloat16
    npad = jnp.concatenate([
        jnp.zeros((B, _M, _FN), f32), input_nodes,
        jnp.zeros((B, pad_tail, _FN), f32)], axis=1).astype(bf16)

    W1 = jnp.concatenate(
        [Wan1, jnp.concatenate([Wae1, jnp.zeros((_FN, 2 * 8), f32)], axis=0)], axis=1)
    b1 = jnp.concatenate([ban1, bae1]).reshape(1, -1)
    W2 = jnp.concatenate([
        jnp.concatenate([Wan2, jnp.zeros((2 * _FN, 8), f32)], axis=1),
        jnp.concatenate([jnp.zeros((2 * 8, _FN), f32), Wae2], axis=1)], axis=0)
    b2 = jnp.concatenate([ban2, bae2]).reshape(1, -1)
    L1a = jnp.concatenate([jnp.zeros((_FN, 12), f32), Wle1[:8]], axis=0)
    N1a = jnp.concatenate([Wln1[:_FN], jnp.zeros((8, 2 * _FN), f32)], axis=0)

    weights = (
        W1.astype(bf16), b1, W2.astype(bf16), b2,
        L1a.astype(bf16), Wle1[8:].astype(bf16), ble1.reshape(1, -1),
        Wle2.astype(bf16), ble2.reshape(1, -1),
        N1a.astype(bf16), Wln1[_FN:].astype(bf16), bln1.reshape(1, -1),
        Wln2.astype(bf16), bln2.reshape(1, -1),
    )

    in_specs = [
        pl.BlockSpec((1, _T, _FN), lambda b, t: (b, t, 0)),
        pl.BlockSpec((1, _T, _FN), lambda b, t: (b, t + 1, 0)),
        pl.BlockSpec((1, _M, _T, _FE), lambda b, t: (b, 0, t, 0)),
        pl.BlockSpec((_M, _T, 1), lambda b, t: (0, t, 0)),
        pl.BlockSpec((_M, _T, 1), lambda b, t: (0, t, 0)),
        pl.BlockSpec((_T, 1), lambda b, t: (t, 0)),
    ] + [_full_spec(w.shape) for w in weights]
    out_specs = [
        pl.BlockSpec((1, _T, _FN), lambda b, t: (b, t, 0)),
        pl.BlockSpec((1, _M, _T, _FE), lambda b, t: (b, 0, t, 0)),
    ]
    out_shape = [
        jax.ShapeDtypeStruct((B, _N, _FN), f32),
        jax.ShapeDtypeStruct((B, _M, _N, _FE), f32),
    ]

    on, oed = pl.pallas_call(
        _graph_body,
        grid=(B, _NT),
        in_specs=in_specs,
        out_specs=out_specs,
        out_shape=out_shape,
        compiler_params=pltpu.CompilerParams(
            dimension_semantics=("parallel", "parallel")),
    )(npad, npad, dense, jnp.asarray(_VALID), jnp.asarray(_NORME),
      jnp.asarray(_NORMN), *weights)

    flat = oed.transpose(0, 2, 1, 3).reshape(B, _N * _M, _FE)
    out_edges = jnp.concatenate([flat[:, _HEAD_OUT, :], flat[:, _HEAD:, :]], axis=1)
    return on, out_edges
